# baseline (speedup 1.0000x reference)
import functools

import jax
import jax.numpy as jnp
from jax import lax
from jax.experimental import pallas as pl
from jax.experimental.pallas import tpu as pltpu
from jax.experimental.pallas import tpu_sc as plsc

N = 10000
E = 320000
D = 128
NC = 2
NS = 16
NW = NC * NS
EPW = E // NW
KB = 80
NBATCH = EPW // KB
G = KB // 16

_mesh = plsc.VectorSubcoreMesh(core_axis_name="c", subcore_axis_name="s")

ROW_BLK = 1000
GRID = N // ROW_BLK


def _dense_tail(h, Wl, bl, beta, lin_ref, rn_ref, rnb_ref, se_ref):
    lin_ref[...] = jnp.dot(h, Wl, preferred_element_type=jnp.float32) + bl
    ss = jnp.sum(h * h, axis=1, keepdims=True)
    nrm = jnp.maximum(jnp.sqrt(ss), 1e-12)
    rn = 1.0 / nrm
    rn_ref[...] = rn
    rnb_ref[...] = rn * beta
    se_ref[...] = jnp.exp(beta * (ss * rn * rn))


def _prep0_body(x_ref, W1_ref, b1_ref, Wl_ref, bl_ref, beta_ref,
                h_ref, lin_ref, rn_ref, rnb_ref, se_ref):
    h = jnp.dot(x_ref[...], W1_ref[...],
                preferred_element_type=jnp.float32) + b1_ref[...]
    h_ref[...] = h
    _dense_tail(h, Wl_ref[...], bl_ref[...], beta_ref[...],
                lin_ref, rn_ref, rnb_ref, se_ref)


def _prep0(x, W1, b1, Wl, bl, beta):
    return pl.pallas_call(
        _prep0_body,
        grid=(GRID,),
        in_specs=[
            pl.BlockSpec((ROW_BLK, D), lambda i: (i, 0)),
            pl.BlockSpec((D, D), lambda i: (0, 0)),
            pl.BlockSpec((1, D), lambda i: (0, 0)),
            pl.BlockSpec((D, D), lambda i: (0, 0)),
            pl.BlockSpec((1, D), lambda i: (0, 0)),
            pl.BlockSpec((1, 1), lambda i: (0, 0)),
        ],
        out_specs=[
            pl.BlockSpec((ROW_BLK, D), lambda i: (i, 0)),
            pl.BlockSpec((ROW_BLK, D), lambda i: (i, 0)),
            pl.BlockSpec((ROW_BLK, 1), lambda i: (i, 0)),
            pl.BlockSpec((ROW_BLK, 1), lambda i: (i, 0)),
            pl.BlockSpec((ROW_BLK, 1), lambda i: (i, 0)),
        ],
        out_shape=[
            jax.ShapeDtypeStruct((N, D), jnp.float32),
            jax.ShapeDtypeStruct((N, D), jnp.float32),
            jax.ShapeDtypeStruct((N, 1), jnp.float32),
            jax.ShapeDtypeStruct((N, 1), jnp.float32),
            jax.ShapeDtypeStruct((N, 1), jnp.float32),
        ],
    )(x, W1, b1.reshape(1, D), Wl, bl.reshape(1, D), beta.reshape(1, 1))


def _mid_body(den_ref, se_ref, rd_ref, sw_ref):
    dsum = jnp.sum(den_ref[...], axis=0, keepdims=True) + se_ref[...]
    rd = 1.0 / (dsum + 1e-16)
    rd_ref[...] = rd
    sw_ref[...] = se_ref[...] * rd


def _mid(denp, se):
    return pl.pallas_call(
        _mid_body,
        out_shape=[
            jax.ShapeDtypeStruct((1, N), jnp.float32),
            jax.ShapeDtypeStruct((1, N), jnp.float32),
        ],
    )(denp, se.reshape(1, N))


def _fused_body(o0_ref, o1_ref, sw_ref, h_ref, lin_ref, Wl_ref, bl_ref,
                beta_ref, h2_ref, lin2_ref, rn_ref, rnb_ref, se_ref):
    z = (o0_ref[...] + o1_ref[...] + sw_ref[...] * h_ref[...]
         + lin_ref[...])
    h = jnp.where(z > 0, z, jnp.exp(z) - 1.0)
    h2_ref[...] = h
    _dense_tail(h, Wl_ref[...], bl_ref[...], beta_ref[...],
                lin2_ref, rn_ref, rnb_ref, se_ref)


def _fused(outp, sw, h, lin, Wl, bl, beta):
    return pl.pallas_call(
        _fused_body,
        grid=(GRID,),
        in_specs=[
            pl.BlockSpec((ROW_BLK, D), lambda i: (i, 0)),
            pl.BlockSpec((ROW_BLK, D), lambda i: (i, 0)),
            pl.BlockSpec((ROW_BLK, 1), lambda i: (i, 0)),
            pl.BlockSpec((ROW_BLK, D), lambda i: (i, 0)),
            pl.BlockSpec((ROW_BLK, D), lambda i: (i, 0)),
            pl.BlockSpec((D, D), lambda i: (0, 0)),
            pl.BlockSpec((1, D), lambda i: (0, 0)),
            pl.BlockSpec((1, 1), lambda i: (0, 0)),
        ],
        out_specs=[
            pl.BlockSpec((ROW_BLK, D), lambda i: (i, 0)),
            pl.BlockSpec((ROW_BLK, D), lambda i: (i, 0)),
            pl.BlockSpec((ROW_BLK, 1), lambda i: (i, 0)),
            pl.BlockSpec((ROW_BLK, 1), lambda i: (i, 0)),
            pl.BlockSpec((ROW_BLK, 1), lambda i: (i, 0)),
        ],
        out_shape=[
            jax.ShapeDtypeStruct((N, D), jnp.float32),
            jax.ShapeDtypeStruct((N, D), jnp.float32),
            jax.ShapeDtypeStruct((N, 1), jnp.float32),
            jax.ShapeDtypeStruct((N, 1), jnp.float32),
            jax.ShapeDtypeStruct((N, 1), jnp.float32),
        ],
    )(outp[0], outp[1],
      sw.reshape(N, 1), h, lin, Wl, bl.reshape(1, D), beta.reshape(1, 1))


def _last_body(o0_ref, o1_ref, sw_ref, h_ref, lin_ref, W4_ref, b4_ref,
               out_ref):
    z = (o0_ref[...] + o1_ref[...] + sw_ref[...] * h_ref[...]
         + lin_ref[...])
    h = jnp.where(z > 0, z, jnp.exp(z) - 1.0)
    out_ref[...] = jnp.dot(h, W4_ref[...],
                           preferred_element_type=jnp.float32) + b4_ref[...]


def _last(outp, sw, h, lin, W4, b4):
    C = W4.shape[1]
    return pl.pallas_call(
        _last_body,
        grid=(GRID,),
        in_specs=[
            pl.BlockSpec((ROW_BLK, D), lambda i: (i, 0)),
            pl.BlockSpec((ROW_BLK, D), lambda i: (i, 0)),
            pl.BlockSpec((ROW_BLK, 1), lambda i: (i, 0)),
            pl.BlockSpec((ROW_BLK, D), lambda i: (i, 0)),
            pl.BlockSpec((ROW_BLK, D), lambda i: (i, 0)),
            pl.BlockSpec((D, C), lambda i: (0, 0)),
            pl.BlockSpec((1, C), lambda i: (0, 0)),
        ],
        out_specs=pl.BlockSpec((ROW_BLK, C), lambda i: (i, 0)),
        out_shape=jax.ShapeDtypeStruct((N, C), jnp.float32),
    )(outp[0], outp[1],
      sw.reshape(N, 1), h, lin, W4, b4.reshape(1, C))


@functools.partial(
    pl.kernel,
    out_type=(
        jax.ShapeDtypeStruct((E,), jnp.float32),
        jax.ShapeDtypeStruct((NW * N,), jnp.float32),
    ),
    mesh=_mesh,
    scratch_types=[
        pltpu.VMEM((EPW,), jnp.int32),
        pltpu.VMEM((EPW,), jnp.int32),
        pltpu.VMEM((N,), jnp.float32),
        pltpu.VMEM((N,), jnp.float32),
        pltpu.VMEM((N,), jnp.float32),
        pltpu.VMEM((EPW,), jnp.float32),
        pltpu.VMEM((2, KB, D), jnp.float32),
        pltpu.VMEM((2, KB, D), jnp.float32),
        pltpu.VMEM((KB * 16,), jnp.float32),
        pltpu.SemaphoreType.DMA((2,)),
        pltpu.SemaphoreType.DMA((2,)),
    ],
    compiler_params=pltpu.CompilerParams(needs_layout_passes=False),
)
def _sc_pass_a(h_hbm, src_hbm, dst_hbm, rnb_hbm, rn_hbm,
               ee_hbm, den_hbm,
               srcall, dstall, rnbt, rnt, denloc, eeall, rowS, rowD, accbuf,
               semS, semD):
    c = lax.axis_index("c")
    s = lax.axis_index("s")
    wid = c * NS + s
    base = wid * EPW

    pltpu.sync_copy(src_hbm.at[pl.ds(base, EPW)], srcall)
    pltpu.sync_copy(dst_hbm.at[pl.ds(base, EPW)], dstall)
    pltpu.sync_copy(rnb_hbm, rnbt)
    pltpu.sync_copy(rn_hbm, rnt)

    def _zero(i, _):
        denloc[pl.ds(i * 16, 16)] = jnp.zeros((16,), jnp.float32)
        return 0
    lax.fori_loop(0, N // 16, _zero, 0)

    def _issue(k, b):
        for g in range(G):
            sv = srcall[pl.ds(k * KB + 16 * g, 16)]
            dv = dstall[pl.ds(k * KB + 16 * g, 16)]
            pltpu.async_copy(h_hbm.at[sv], rowS.at[b, pl.ds(16 * g, 16)],
                             semS.at[b])
            pltpu.async_copy(h_hbm.at[dv], rowD.at[b, pl.ds(16 * g, 16)],
                             semD.at[b])

    _issue(0, 0)

    def _batch(bk, _):
        b = lax.rem(bk, 2)

        @pl.when(bk + 1 < NBATCH)
        def _():
            _issue(bk + 1, 1 - b)

        pltpu.make_async_copy(h_hbm.at[pl.ds(0, KB)], rowS.at[b],
                              semS.at[b]).wait()
        pltpu.make_async_copy(h_hbm.at[pl.ds(0, KB)], rowD.at[b],
                              semD.at[b]).wait()

        io = lax.iota(jnp.int32, 16)

        @plsc.parallel_loop(0, KB, unroll=4)
        def _edge(e):
            a0 = rowS[b, e, pl.ds(0, 16)] * rowD[b, e, pl.ds(0, 16)]
            a1 = rowS[b, e, pl.ds(16, 16)] * rowD[b, e, pl.ds(16, 16)]
            for t in range(2, 8, 2):
                a0 += (rowS[b, e, pl.ds(16 * t, 16)]
                       * rowD[b, e, pl.ds(16 * t, 16)])
                a1 += (rowS[b, e, pl.ds(16 * (t + 1), 16)]
                       * rowD[b, e, pl.ds(16 * (t + 1), 16)])
            accbuf[pl.ds(e * 16, 16)] = a0 + a1

        for g in range(G):
            idxb = (io + 16 * g) * 16
            vals = [plsc.load_gather(accbuf, [idxb + l]) for l in range(16)]
            while len(vals) > 1:
                vals = [vals[i] + vals[i + 1] for i in range(0, len(vals), 2)]
            acc = vals[0]
            sv = srcall[pl.ds(bk * KB + 16 * g, 16)]
            dv = dstall[pl.ds(bk * KB + 16 * g, 16)]
            rns = plsc.load_gather(rnbt, [sv])
            rnd = plsc.load_gather(rnt, [dv])
            eev = jnp.exp(acc * rns * rnd)
            eeall[pl.ds(bk * KB + 16 * g, 16)] = eev

            kv, vv = plsc.sort_key_val(dv, eev)
            csum = plsc.cumsum(vv)
            knext = jnp.take_along_axis(kv, jnp.minimum(io + 1, 15), axis=0)
            last = (kv != knext) | (io == 15)
            kprev = jnp.take_along_axis(kv, jnp.maximum(io - 1, 0), axis=0)
            first = (kv != kprev) | (io == 0)
            sor = plsc.cummax(jnp.where(first, io, 0))
            prev_cs = jnp.take_along_axis(csum, jnp.maximum(sor - 1, 0),
                                          axis=0)
            runsum = csum - jnp.where(sor > 0, prev_cs, 0.0)
            plsc.addupdate_scatter(denloc, [kv], runsum, mask=last)
        return 0

    lax.fori_loop(0, NBATCH, _batch, 0)
    pltpu.sync_copy(eeall, ee_hbm.at[pl.ds(base, EPW)])
    pltpu.sync_copy(denloc, den_hbm.at[pl.ds(wid * N, N)])


KB2 = 32
G2 = KB2 // 16
NB2 = (EPW + KB2 - 1) // KB2
EPAD = NB2 * KB2
RB = 624


@functools.partial(
    pl.kernel,
    out_type=jax.ShapeDtypeStruct((NC, N, D), jnp.float32),
    mesh=_mesh,
    scratch_types=[
        pltpu.VMEM((EPAD,), jnp.int32),
        pltpu.VMEM((EPAD,), jnp.int32),
        pltpu.VMEM((N,), jnp.float32),
        pltpu.VMEM((EPAD,), jnp.float32),
        pltpu.VMEM((KB2,), jnp.float32),
        pltpu.VMEM((2, KB2, D), jnp.float32),
        pltpu.VMEM_SHARED((N, D), jnp.float32),
        pltpu.SemaphoreType.DMA((2,)),
        pltpu.SemaphoreType.DMA((2,)),
    ],
    compiler_params=pltpu.CompilerParams(needs_layout_passes=False),
)
def _sc_pass_b(h_hbm, src_hbm, dst_hbm, ee_hbm, rd_hbm, zz_hbm,
               out_hbm,
               srcall, dstall, rdt, eeall, wbuf, rowS, spacc, semS, semW):
    c = lax.axis_index("c")
    s = lax.axis_index("s")
    wid = c * NS + s
    base = wid * EPW

    pltpu.sync_copy(src_hbm.at[pl.ds(base, EPW)], srcall.at[pl.ds(0, EPW)])
    pltpu.sync_copy(dst_hbm.at[pl.ds(base, EPW)], dstall.at[pl.ds(0, EPW)])
    pltpu.sync_copy(ee_hbm.at[pl.ds(base, EPW)], eeall.at[pl.ds(0, EPW)])
    pltpu.sync_copy(rd_hbm, rdt)
    srcall[pl.ds(EPW, EPAD - EPW)] = jnp.zeros((EPAD - EPW,), jnp.int32)
    dstall[pl.ds(EPW, EPAD - EPW)] = jnp.zeros((EPAD - EPW,), jnp.int32)

    nz = lax.select(s == NS - 1, 40, 39)
    rbase = s * RB

    def _zchunk(i, _):
        pltpu.sync_copy(zz_hbm, spacc.at[pl.ds(rbase + 16 * i, 16)])
        return 0
    lax.fori_loop(0, nz, _zchunk, 0)
    plsc.subcore_barrier()

    io = lax.iota(jnp.int32, 16)

    def _issue(k, b):
        for g in range(G2):
            sv = srcall[pl.ds(k * KB2 + 16 * g, 16)]
            pltpu.async_copy(h_hbm.at[sv], rowS.at[b, pl.ds(16 * g, 16)],
                             semS.at[b])

    _issue(0, 0)

    def _drain_scatter(b):
        for _ in range(G2):
            pltpu.make_async_copy(h_hbm.at[pl.ds(0, 16)],
                                  rowS.at[b, pl.ds(0, 16)],
                                  semW.at[b]).wait()

    def _batch(bk, _):
        b = lax.rem(bk, 2)

        @pl.when(bk >= 1)
        def _():
            _drain_scatter(1 - b)

        @pl.when(bk + 1 < NB2)
        def _():
            _issue(bk + 1, 1 - b)

        pltpu.make_async_copy(h_hbm.at[pl.ds(0, KB2)], rowS.at[b],
                              semS.at[b]).wait()

        for g in range(G2):
            off = bk * KB2 + 16 * g
            dv = dstall[pl.ds(off, 16)]
            rdv = plsc.load_gather(rdt, [dv])
            w = eeall[pl.ds(off, 16)] * rdv
            wbuf[pl.ds(16 * g, 16)] = jnp.where(off + io < EPW, w, 0.0)

        @plsc.parallel_loop(0, KB2, unroll=2)
        def _scale(e):
            wv = plsc.load_gather(wbuf, [jnp.full((16,), 0, jnp.int32) + e])
            for t in range(8):
                rowS[b, e, pl.ds(16 * t, 16)] = (
                    rowS[b, e, pl.ds(16 * t, 16)] * wv)

        for g in range(G2):
            dv = dstall[pl.ds(bk * KB2 + 16 * g, 16)]
            pltpu.async_copy(rowS.at[b, pl.ds(16 * g, 16)], spacc.at[dv],
                             semW.at[b], add=True)
        return 0

    lax.fori_loop(0, NB2, _batch, 0)
    _drain_scatter(lax.rem(NB2 - 1, 2))
    plsc.subcore_barrier()

    def _ochunk(i, _):
        pltpu.sync_copy(spacc.at[pl.ds(rbase + 16 * i, 16)],
                        out_hbm.at[c, pl.ds(rbase + 16 * i, 16)])
        return 0
    lax.fori_loop(0, nz, _ochunk, 0)


def kernel(x, edge_index, W1, b1, Wl1, bl1, Wl2, bl2, Wl3, bl3, Wl4, bl4,
           W4, b4, beta2, beta3, beta5, beta6):
    src = edge_index[0]
    dst = edge_index[1]
    zz = jnp.zeros((16, D), jnp.float32)

    h, lin, rn, rnb, se = _prep0(x, W1, b1, Wl1, bl1, beta2)

    layers = [(Wl2, bl2, beta3), (Wl3, bl3, beta5), (Wl4, bl4, beta6)]
    for i in range(4):
        ee, denp = _sc_pass_a(h, src, dst, rnb.reshape(N), rn.reshape(N))
        rd, sw = _mid(denp.reshape(NW, N), se)
        outp = _sc_pass_b(h, src, dst, ee, rd.reshape(N), zz)
        if i < 3:
            Wl, bl, beta = layers[i]
            h, lin, rn, rnb, se = _fused(outp, sw, h, lin, Wl, bl, beta)
        else:
            return _last(outp, sw, h, lin, W4, b4)


# speedup vs baseline: 1.0280x; 1.0280x over previous
import functools

import jax
import jax.numpy as jnp
from jax import lax
from jax.experimental import pallas as pl
from jax.experimental.pallas import tpu as pltpu
from jax.experimental.pallas import tpu_sc as plsc

N = 10000
E = 320000
D = 128
NC = 2
NS = 16
NW = NC * NS
EPW = E // NW
KB = 80
NBATCH = EPW // KB
G = KB // 16

_mesh = plsc.VectorSubcoreMesh(core_axis_name="c", subcore_axis_name="s")

ROW_BLK = 1000
GRID = N // ROW_BLK


def _dense_tail(h, Wl, bl, beta, lin_ref, rn_ref, rnb_ref, se_ref):
    lin_ref[...] = jnp.dot(h, Wl, preferred_element_type=jnp.float32) + bl
    ss = jnp.sum(h * h, axis=1, keepdims=True)
    nrm = jnp.maximum(jnp.sqrt(ss), 1e-12)
    rn = 1.0 / nrm
    rn_ref[...] = rn
    rnb_ref[...] = rn * beta
    se_ref[...] = jnp.exp(beta * (ss * rn * rn))


def _prep0_body(x_ref, W1_ref, b1_ref, Wl_ref, bl_ref, beta_ref,
                h_ref, lin_ref, rn_ref, rnb_ref, se_ref):
    h = jnp.dot(x_ref[...], W1_ref[...],
                preferred_element_type=jnp.float32) + b1_ref[...]
    h_ref[...] = h
    _dense_tail(h, Wl_ref[...], bl_ref[...], beta_ref[...],
                lin_ref, rn_ref, rnb_ref, se_ref)


def _prep0(x, W1, b1, Wl, bl, beta):
    return pl.pallas_call(
        _prep0_body,
        grid=(GRID,),
        in_specs=[
            pl.BlockSpec((ROW_BLK, D), lambda i: (i, 0)),
            pl.BlockSpec((D, D), lambda i: (0, 0)),
            pl.BlockSpec((1, D), lambda i: (0, 0)),
            pl.BlockSpec((D, D), lambda i: (0, 0)),
            pl.BlockSpec((1, D), lambda i: (0, 0)),
            pl.BlockSpec((1, 1), lambda i: (0, 0)),
        ],
        out_specs=[
            pl.BlockSpec((ROW_BLK, D), lambda i: (i, 0)),
            pl.BlockSpec((ROW_BLK, D), lambda i: (i, 0)),
            pl.BlockSpec((ROW_BLK, 1), lambda i: (i, 0)),
            pl.BlockSpec((ROW_BLK, 1), lambda i: (i, 0)),
            pl.BlockSpec((ROW_BLK, 1), lambda i: (i, 0)),
        ],
        out_shape=[
            jax.ShapeDtypeStruct((N, D), jnp.float32),
            jax.ShapeDtypeStruct((N, D), jnp.float32),
            jax.ShapeDtypeStruct((N, 1), jnp.float32),
            jax.ShapeDtypeStruct((N, 1), jnp.float32),
            jax.ShapeDtypeStruct((N, 1), jnp.float32),
        ],
    )(x, W1, b1.reshape(1, D), Wl, bl.reshape(1, D), beta.reshape(1, 1))


def _mid_body(den_ref, se_ref, rd_ref, sw_ref):
    dsum = jnp.sum(den_ref[...], axis=0, keepdims=True) + se_ref[...]
    rd = 1.0 / (dsum + 1e-16)
    rd_ref[...] = rd
    sw_ref[...] = se_ref[...] * rd


def _mid(denp, se):
    return pl.pallas_call(
        _mid_body,
        out_shape=[
            jax.ShapeDtypeStruct((1, N), jnp.float32),
            jax.ShapeDtypeStruct((1, N), jnp.float32),
        ],
    )(denp, se.reshape(1, N))


def _fused_body(o0_ref, o1_ref, sw_ref, h_ref, lin_ref, Wl_ref, bl_ref,
                beta_ref, h2_ref, lin2_ref, rn_ref, rnb_ref, se_ref):
    z = (o0_ref[...] + o1_ref[...] + sw_ref[...] * h_ref[...]
         + lin_ref[...])
    h = jnp.where(z > 0, z, jnp.exp(z) - 1.0)
    h2_ref[...] = h
    _dense_tail(h, Wl_ref[...], bl_ref[...], beta_ref[...],
                lin2_ref, rn_ref, rnb_ref, se_ref)


def _fused(outp, sw, h, lin, Wl, bl, beta):
    return pl.pallas_call(
        _fused_body,
        grid=(GRID,),
        in_specs=[
            pl.BlockSpec((ROW_BLK, D), lambda i: (i, 0)),
            pl.BlockSpec((ROW_BLK, D), lambda i: (i, 0)),
            pl.BlockSpec((ROW_BLK, 1), lambda i: (i, 0)),
            pl.BlockSpec((ROW_BLK, D), lambda i: (i, 0)),
            pl.BlockSpec((ROW_BLK, D), lambda i: (i, 0)),
            pl.BlockSpec((D, D), lambda i: (0, 0)),
            pl.BlockSpec((1, D), lambda i: (0, 0)),
            pl.BlockSpec((1, 1), lambda i: (0, 0)),
        ],
        out_specs=[
            pl.BlockSpec((ROW_BLK, D), lambda i: (i, 0)),
            pl.BlockSpec((ROW_BLK, D), lambda i: (i, 0)),
            pl.BlockSpec((ROW_BLK, 1), lambda i: (i, 0)),
            pl.BlockSpec((ROW_BLK, 1), lambda i: (i, 0)),
            pl.BlockSpec((ROW_BLK, 1), lambda i: (i, 0)),
        ],
        out_shape=[
            jax.ShapeDtypeStruct((N, D), jnp.float32),
            jax.ShapeDtypeStruct((N, D), jnp.float32),
            jax.ShapeDtypeStruct((N, 1), jnp.float32),
            jax.ShapeDtypeStruct((N, 1), jnp.float32),
            jax.ShapeDtypeStruct((N, 1), jnp.float32),
        ],
    )(outp[0], outp[1],
      sw.reshape(N, 1), h, lin, Wl, bl.reshape(1, D), beta.reshape(1, 1))


def _last_body(o0_ref, o1_ref, sw_ref, h_ref, lin_ref, W4_ref, b4_ref,
               out_ref):
    z = (o0_ref[...] + o1_ref[...] + sw_ref[...] * h_ref[...]
         + lin_ref[...])
    h = jnp.where(z > 0, z, jnp.exp(z) - 1.0)
    out_ref[...] = jnp.dot(h, W4_ref[...],
                           preferred_element_type=jnp.float32) + b4_ref[...]


def _last(outp, sw, h, lin, W4, b4):
    C = W4.shape[1]
    return pl.pallas_call(
        _last_body,
        grid=(GRID,),
        in_specs=[
            pl.BlockSpec((ROW_BLK, D), lambda i: (i, 0)),
            pl.BlockSpec((ROW_BLK, D), lambda i: (i, 0)),
            pl.BlockSpec((ROW_BLK, 1), lambda i: (i, 0)),
            pl.BlockSpec((ROW_BLK, D), lambda i: (i, 0)),
            pl.BlockSpec((ROW_BLK, D), lambda i: (i, 0)),
            pl.BlockSpec((D, C), lambda i: (0, 0)),
            pl.BlockSpec((1, C), lambda i: (0, 0)),
        ],
        out_specs=pl.BlockSpec((ROW_BLK, C), lambda i: (i, 0)),
        out_shape=jax.ShapeDtypeStruct((N, C), jnp.float32),
    )(outp[0], outp[1],
      sw.reshape(N, 1), h, lin, W4, b4.reshape(1, C))


@functools.partial(
    pl.kernel,
    out_type=(
        jax.ShapeDtypeStruct((E,), jnp.float32),
        jax.ShapeDtypeStruct((NW * N,), jnp.float32),
    ),
    mesh=_mesh,
    scratch_types=[
        pltpu.VMEM((EPW,), jnp.int32),
        pltpu.VMEM((EPW,), jnp.int32),
        pltpu.VMEM((N,), jnp.float32),
        pltpu.VMEM((N,), jnp.float32),
        pltpu.VMEM((N,), jnp.float32),
        pltpu.VMEM((EPW,), jnp.float32),
        pltpu.VMEM((2, KB, D // 2), jnp.float32),
        pltpu.VMEM((2, KB, D // 2), jnp.float32),
        pltpu.VMEM((KB * 16,), jnp.float32),
        pltpu.SemaphoreType.DMA((2,)),
        pltpu.SemaphoreType.DMA((2,)),
    ],
    compiler_params=pltpu.CompilerParams(needs_layout_passes=False,
                                         use_tc_tiling_on_sc=False),
)
def _sc_pass_a(h_hbm, src_hbm, dst_hbm, rnb_hbm, rn_hbm,
               ee_hbm, den_hbm,
               srcall, dstall, rnbt, rnt, denloc, eeall, rowS, rowD, accbuf,
               semS, semD):
    c = lax.axis_index("c")
    s = lax.axis_index("s")
    wid = c * NS + s
    base = wid * EPW

    pltpu.sync_copy(src_hbm.at[pl.ds(base, EPW)], srcall)
    pltpu.sync_copy(dst_hbm.at[pl.ds(base, EPW)], dstall)
    pltpu.sync_copy(rnb_hbm, rnbt)
    pltpu.sync_copy(rn_hbm, rnt)

    def _zero(i, _):
        denloc[pl.ds(i * 16, 16)] = jnp.zeros((16,), jnp.float32)
        return 0
    lax.fori_loop(0, N // 16, _zero, 0)

    def _issue(k, b):
        for g in range(G):
            sv = srcall[pl.ds(k * KB + 16 * g, 16)]
            dv = dstall[pl.ds(k * KB + 16 * g, 16)]
            pltpu.async_copy(h_hbm.at[sv], rowS.at[b, pl.ds(16 * g, 16)],
                             semS.at[b])
            pltpu.async_copy(h_hbm.at[dv], rowD.at[b, pl.ds(16 * g, 16)],
                             semD.at[b])

    _issue(0, 0)

    def _batch(bk, _):
        b = lax.rem(bk, 2)

        @pl.when(bk + 1 < NBATCH)
        def _():
            _issue(bk + 1, 1 - b)

        pltpu.make_async_copy(h_hbm.at[pl.ds(0, KB)], rowS.at[b],
                              semS.at[b]).wait()
        pltpu.make_async_copy(h_hbm.at[pl.ds(0, KB)], rowD.at[b],
                              semD.at[b]).wait()

        io = lax.iota(jnp.int32, 16)

        @plsc.parallel_loop(0, KB, unroll=2)
        def _edge(e):
            a0 = jnp.zeros((16,), jnp.float32)
            a1 = jnp.zeros((16,), jnp.float32)
            for t in range(4):
                sa, sb = plsc.unpack(
                    plsc.bitcast(rowS[b, e, pl.ds(16 * t, 16)], jnp.bfloat16),
                    format=plsc.PackFormat.INTERLEAVED)
                da, db = plsc.unpack(
                    plsc.bitcast(rowD[b, e, pl.ds(16 * t, 16)], jnp.bfloat16),
                    format=plsc.PackFormat.INTERLEAVED)
                a0 += sa * da
                a1 += sb * db
            accbuf[pl.ds(e * 16, 16)] = a0 + a1

        for g in range(G):
            idxb = (io + 16 * g) * 16
            vals = [plsc.load_gather(accbuf, [idxb + l]) for l in range(16)]
            while len(vals) > 1:
                vals = [vals[i] + vals[i + 1] for i in range(0, len(vals), 2)]
            acc = vals[0]
            sv = srcall[pl.ds(bk * KB + 16 * g, 16)]
            dv = dstall[pl.ds(bk * KB + 16 * g, 16)]
            rns = plsc.load_gather(rnbt, [sv])
            rnd = plsc.load_gather(rnt, [dv])
            eev = jnp.exp(acc * rns * rnd)
            eeall[pl.ds(bk * KB + 16 * g, 16)] = eev

            kv, vv = plsc.sort_key_val(dv, eev)
            csum = plsc.cumsum(vv)
            knext = jnp.take_along_axis(kv, jnp.minimum(io + 1, 15), axis=0)
            last = (kv != knext) | (io == 15)
            kprev = jnp.take_along_axis(kv, jnp.maximum(io - 1, 0), axis=0)
            first = (kv != kprev) | (io == 0)
            sor = plsc.cummax(jnp.where(first, io, 0))
            prev_cs = jnp.take_along_axis(csum, jnp.maximum(sor - 1, 0),
                                          axis=0)
            runsum = csum - jnp.where(sor > 0, prev_cs, 0.0)
            plsc.addupdate_scatter(denloc, [kv], runsum, mask=last)
        return 0

    lax.fori_loop(0, NBATCH, _batch, 0)
    pltpu.sync_copy(eeall, ee_hbm.at[pl.ds(base, EPW)])
    pltpu.sync_copy(denloc, den_hbm.at[pl.ds(wid * N, N)])


KB2 = 32
G2 = KB2 // 16
NB2 = (EPW + KB2 - 1) // KB2
EPAD = NB2 * KB2
RB = 624


@functools.partial(
    pl.kernel,
    out_type=jax.ShapeDtypeStruct((NC, N, D), jnp.float32),
    mesh=_mesh,
    scratch_types=[
        pltpu.VMEM((EPAD,), jnp.int32),
        pltpu.VMEM((EPAD,), jnp.int32),
        pltpu.VMEM((N,), jnp.float32),
        pltpu.VMEM((EPAD,), jnp.float32),
        pltpu.VMEM((KB2,), jnp.float32),
        pltpu.VMEM((2, KB2, D), jnp.float32),
        pltpu.VMEM_SHARED((N, D), jnp.float32),
        pltpu.SemaphoreType.DMA((2,)),
        pltpu.SemaphoreType.DMA((2,)),
    ],
    compiler_params=pltpu.CompilerParams(needs_layout_passes=False),
)
def _sc_pass_b(h_hbm, src_hbm, dst_hbm, ee_hbm, rd_hbm, zz_hbm,
               out_hbm,
               srcall, dstall, rdt, eeall, wbuf, rowS, spacc, semS, semW):
    c = lax.axis_index("c")
    s = lax.axis_index("s")
    wid = c * NS + s
    base = wid * EPW

    pltpu.sync_copy(src_hbm.at[pl.ds(base, EPW)], srcall.at[pl.ds(0, EPW)])
    pltpu.sync_copy(dst_hbm.at[pl.ds(base, EPW)], dstall.at[pl.ds(0, EPW)])
    pltpu.sync_copy(ee_hbm.at[pl.ds(base, EPW)], eeall.at[pl.ds(0, EPW)])
    pltpu.sync_copy(rd_hbm, rdt)
    srcall[pl.ds(EPW, EPAD - EPW)] = jnp.zeros((EPAD - EPW,), jnp.int32)
    dstall[pl.ds(EPW, EPAD - EPW)] = jnp.zeros((EPAD - EPW,), jnp.int32)

    nz = lax.select(s == NS - 1, 40, 39)
    rbase = s * RB

    def _zchunk(i, _):
        pltpu.sync_copy(zz_hbm, spacc.at[pl.ds(rbase + 16 * i, 16)])
        return 0
    lax.fori_loop(0, nz, _zchunk, 0)
    plsc.subcore_barrier()

    io = lax.iota(jnp.int32, 16)

    def _issue(k, b):
        for g in range(G2):
            sv = srcall[pl.ds(k * KB2 + 16 * g, 16)]
            pltpu.async_copy(h_hbm.at[sv], rowS.at[b, pl.ds(16 * g, 16)],
                             semS.at[b])

    _issue(0, 0)

    def _drain_scatter(b):
        for _ in range(G2):
            pltpu.make_async_copy(h_hbm.at[pl.ds(0, 16)],
                                  rowS.at[b, pl.ds(0, 16)],
                                  semW.at[b]).wait()

    def _batch(bk, _):
        b = lax.rem(bk, 2)

        @pl.when(bk >= 1)
        def _():
            _drain_scatter(1 - b)

        @pl.when(bk + 1 < NB2)
        def _():
            _issue(bk + 1, 1 - b)

        pltpu.make_async_copy(h_hbm.at[pl.ds(0, KB2)], rowS.at[b],
                              semS.at[b]).wait()

        for g in range(G2):
            off = bk * KB2 + 16 * g
            dv = dstall[pl.ds(off, 16)]
            rdv = plsc.load_gather(rdt, [dv])
            w = eeall[pl.ds(off, 16)] * rdv
            wbuf[pl.ds(16 * g, 16)] = jnp.where(off + io < EPW, w, 0.0)

        @plsc.parallel_loop(0, KB2, unroll=2)
        def _scale(e):
            wv = plsc.load_gather(wbuf, [jnp.full((16,), 0, jnp.int32) + e])
            for t in range(8):
                rowS[b, e, pl.ds(16 * t, 16)] = (
                    rowS[b, e, pl.ds(16 * t, 16)] * wv)

        for g in range(G2):
            dv = dstall[pl.ds(bk * KB2 + 16 * g, 16)]
            pltpu.async_copy(rowS.at[b, pl.ds(16 * g, 16)], spacc.at[dv],
                             semW.at[b], add=True)
        return 0

    lax.fori_loop(0, NB2, _batch, 0)
    _drain_scatter(lax.rem(NB2 - 1, 2))
    plsc.subcore_barrier()

    def _ochunk(i, _):
        pltpu.sync_copy(spacc.at[pl.ds(rbase + 16 * i, 16)],
                        out_hbm.at[c, pl.ds(rbase + 16 * i, 16)])
        return 0
    lax.fori_loop(0, nz, _ochunk, 0)


def kernel(x, edge_index, W1, b1, Wl1, bl1, Wl2, bl2, Wl3, bl3, Wl4, bl4,
           W4, b4, beta2, beta3, beta5, beta6):
    src = edge_index[0]
    dst = edge_index[1]
    zz = jnp.zeros((16, D), jnp.float32)

    h, lin, rn, rnb, se = _prep0(x, W1, b1, Wl1, bl1, beta2)

    layers = [(Wl2, bl2, beta3), (Wl3, bl3, beta5), (Wl4, bl4, beta6)]
    for i in range(4):
        hb = lax.bitcast_convert_type(
            h.astype(jnp.bfloat16).reshape(N, D // 2, 2), jnp.float32)
        ee, denp = _sc_pass_a(hb, src, dst, rnb.reshape(N), rn.reshape(N))
        rd, sw = _mid(denp.reshape(NW, N), se)
        outp = _sc_pass_b(h, src, dst, ee, rd.reshape(N), zz)
        if i < 3:
            Wl, bl, beta = layers[i]
            h, lin, rn, rnb, se = _fused(outp, sw, h, lin, Wl, bl, beta)
        else:
            return _last(outp, sw, h, lin, W4, b4)


# speedup vs baseline: 1.0447x; 1.0162x over previous
import functools

import jax
import jax.numpy as jnp
from jax import lax
from jax.experimental import pallas as pl
from jax.experimental.pallas import tpu as pltpu
from jax.experimental.pallas import tpu_sc as plsc

N = 10000
E = 320000
D = 128
NC = 2
NS = 16
NW = NC * NS
EPW = E // NW
KB = 80
NBATCH = EPW // KB
G = KB // 16

_mesh = plsc.VectorSubcoreMesh(core_axis_name="c", subcore_axis_name="s")

ROW_BLK = 1000
GRID = N // ROW_BLK


def _dense_tail(h, Wl, bl, beta, lin_ref, rn_ref, rnb_ref, se_ref):
    lin_ref[...] = jnp.dot(h, Wl, preferred_element_type=jnp.float32) + bl
    ss = jnp.sum(h * h, axis=1, keepdims=True)
    nrm = jnp.maximum(jnp.sqrt(ss), 1e-12)
    rn = 1.0 / nrm
    rn_ref[...] = rn
    rnb_ref[...] = rn * beta
    se_ref[...] = jnp.exp(beta * (ss * rn * rn))


def _prep0_body(x_ref, W1_ref, b1_ref, Wl_ref, bl_ref, beta_ref,
                h_ref, lin_ref, rn_ref, rnb_ref, se_ref):
    h = jnp.dot(x_ref[...], W1_ref[...],
                preferred_element_type=jnp.float32) + b1_ref[...]
    h_ref[...] = h
    _dense_tail(h, Wl_ref[...], bl_ref[...], beta_ref[...],
                lin_ref, rn_ref, rnb_ref, se_ref)


def _prep0(x, W1, b1, Wl, bl, beta):
    return pl.pallas_call(
        _prep0_body,
        grid=(GRID,),
        in_specs=[
            pl.BlockSpec((ROW_BLK, D), lambda i: (i, 0)),
            pl.BlockSpec((D, D), lambda i: (0, 0)),
            pl.BlockSpec((1, D), lambda i: (0, 0)),
            pl.BlockSpec((D, D), lambda i: (0, 0)),
            pl.BlockSpec((1, D), lambda i: (0, 0)),
            pl.BlockSpec((1, 1), lambda i: (0, 0)),
        ],
        out_specs=[
            pl.BlockSpec((ROW_BLK, D), lambda i: (i, 0)),
            pl.BlockSpec((ROW_BLK, D), lambda i: (i, 0)),
            pl.BlockSpec((ROW_BLK, 1), lambda i: (i, 0)),
            pl.BlockSpec((ROW_BLK, 1), lambda i: (i, 0)),
            pl.BlockSpec((ROW_BLK, 1), lambda i: (i, 0)),
        ],
        out_shape=[
            jax.ShapeDtypeStruct((N, D), jnp.float32),
            jax.ShapeDtypeStruct((N, D), jnp.float32),
            jax.ShapeDtypeStruct((N, 1), jnp.float32),
            jax.ShapeDtypeStruct((N, 1), jnp.float32),
            jax.ShapeDtypeStruct((N, 1), jnp.float32),
        ],
    )(x, W1, b1.reshape(1, D), Wl, bl.reshape(1, D), beta.reshape(1, 1))


def _mid_body(den_ref, se_ref, rd_ref, sw_ref):
    dsum = jnp.sum(den_ref[...], axis=0, keepdims=True) + se_ref[...]
    rd = 1.0 / (dsum + 1e-16)
    rd_ref[...] = rd
    sw_ref[...] = se_ref[...] * rd


def _mid(denp, se):
    return pl.pallas_call(
        _mid_body,
        out_shape=[
            jax.ShapeDtypeStruct((1, N), jnp.float32),
            jax.ShapeDtypeStruct((1, N), jnp.float32),
        ],
    )(denp, se.reshape(1, N))


def _fused_body(o0_ref, o1_ref, dnp_ref, se_ref0, h_ref, lin_ref, Wl_ref,
                bl_ref, beta_ref, h2_ref, lin2_ref, rn_ref, rnb_ref, se_ref):
    se0 = se_ref0[...]
    rd = 1.0 / (jnp.sum(dnp_ref[...], axis=1, keepdims=True) + se0)
    z = ((o0_ref[...] + o1_ref[...]) * rd + (se0 * rd) * h_ref[...]
         + lin_ref[...])
    h = jnp.where(z > 0, z, jnp.exp(z) - 1.0)
    h2_ref[...] = h
    _dense_tail(h, Wl_ref[...], bl_ref[...], beta_ref[...],
                lin2_ref, rn_ref, rnb_ref, se_ref)


def _fused(outp, dnpT, se, h, lin, Wl, bl, beta):
    return pl.pallas_call(
        _fused_body,
        grid=(GRID,),
        in_specs=[
            pl.BlockSpec((ROW_BLK, D), lambda i: (i, 0)),
            pl.BlockSpec((ROW_BLK, D), lambda i: (i, 0)),
            pl.BlockSpec((ROW_BLK, NW), lambda i: (i, 0)),
            pl.BlockSpec((ROW_BLK, 1), lambda i: (i, 0)),
            pl.BlockSpec((ROW_BLK, D), lambda i: (i, 0)),
            pl.BlockSpec((ROW_BLK, D), lambda i: (i, 0)),
            pl.BlockSpec((D, D), lambda i: (0, 0)),
            pl.BlockSpec((1, D), lambda i: (0, 0)),
            pl.BlockSpec((1, 1), lambda i: (0, 0)),
        ],
        out_specs=[
            pl.BlockSpec((ROW_BLK, D), lambda i: (i, 0)),
            pl.BlockSpec((ROW_BLK, D), lambda i: (i, 0)),
            pl.BlockSpec((ROW_BLK, 1), lambda i: (i, 0)),
            pl.BlockSpec((ROW_BLK, 1), lambda i: (i, 0)),
            pl.BlockSpec((ROW_BLK, 1), lambda i: (i, 0)),
        ],
        out_shape=[
            jax.ShapeDtypeStruct((N, D), jnp.float32),
            jax.ShapeDtypeStruct((N, D), jnp.float32),
            jax.ShapeDtypeStruct((N, 1), jnp.float32),
            jax.ShapeDtypeStruct((N, 1), jnp.float32),
            jax.ShapeDtypeStruct((N, 1), jnp.float32),
        ],
    )(outp[0], outp[1], dnpT,
      se, h, lin, Wl, bl.reshape(1, D), beta.reshape(1, 1))


def _last_body(o0_ref, o1_ref, dnp_ref, se_ref0, h_ref, lin_ref, W4_ref,
               b4_ref, out_ref):
    se0 = se_ref0[...]
    rd = 1.0 / (jnp.sum(dnp_ref[...], axis=1, keepdims=True) + se0)
    z = ((o0_ref[...] + o1_ref[...]) * rd + (se0 * rd) * h_ref[...]
         + lin_ref[...])
    h = jnp.where(z > 0, z, jnp.exp(z) - 1.0)
    out_ref[...] = jnp.dot(h, W4_ref[...],
                           preferred_element_type=jnp.float32) + b4_ref[...]


def _last(outp, dnpT, se, h, lin, W4, b4):
    C = W4.shape[1]
    return pl.pallas_call(
        _last_body,
        grid=(GRID,),
        in_specs=[
            pl.BlockSpec((ROW_BLK, D), lambda i: (i, 0)),
            pl.BlockSpec((ROW_BLK, D), lambda i: (i, 0)),
            pl.BlockSpec((ROW_BLK, NW), lambda i: (i, 0)),
            pl.BlockSpec((ROW_BLK, 1), lambda i: (i, 0)),
            pl.BlockSpec((ROW_BLK, D), lambda i: (i, 0)),
            pl.BlockSpec((ROW_BLK, D), lambda i: (i, 0)),
            pl.BlockSpec((D, C), lambda i: (0, 0)),
            pl.BlockSpec((1, C), lambda i: (0, 0)),
        ],
        out_specs=pl.BlockSpec((ROW_BLK, C), lambda i: (i, 0)),
        out_shape=jax.ShapeDtypeStruct((N, C), jnp.float32),
    )(outp[0], outp[1], dnpT,
      se, h, lin, W4, b4.reshape(1, C))


@functools.partial(
    pl.kernel,
    out_type=(
        jax.ShapeDtypeStruct((E,), jnp.float32),
        jax.ShapeDtypeStruct((NW * N,), jnp.float32),
    ),
    mesh=_mesh,
    scratch_types=[
        pltpu.VMEM((EPW,), jnp.int32),
        pltpu.VMEM((EPW,), jnp.int32),
        pltpu.VMEM((N,), jnp.float32),
        pltpu.VMEM((N,), jnp.float32),
        pltpu.VMEM((N,), jnp.float32),
        pltpu.VMEM((EPW,), jnp.float32),
        pltpu.VMEM((2, KB, D // 2), jnp.float32),
        pltpu.VMEM((2, KB, D // 2), jnp.float32),
        pltpu.VMEM((KB * 16,), jnp.float32),
        pltpu.SemaphoreType.DMA((2,)),
        pltpu.SemaphoreType.DMA((2,)),
    ],
    compiler_params=pltpu.CompilerParams(needs_layout_passes=False,
                                         use_tc_tiling_on_sc=False),
)
def _sc_pass_a(h_hbm, src_hbm, dst_hbm, rnb_hbm, rn_hbm,
               ee_hbm, den_hbm,
               srcall, dstall, rnbt, rnt, denloc, eeall, rowS, rowD, accbuf,
               semS, semD):
    c = lax.axis_index("c")
    s = lax.axis_index("s")
    wid = c * NS + s
    base = wid * EPW

    pltpu.sync_copy(src_hbm.at[pl.ds(base, EPW)], srcall)
    pltpu.sync_copy(dst_hbm.at[pl.ds(base, EPW)], dstall)
    pltpu.sync_copy(rnb_hbm, rnbt)
    pltpu.sync_copy(rn_hbm, rnt)

    def _zero(i, _):
        denloc[pl.ds(i * 16, 16)] = jnp.zeros((16,), jnp.float32)
        return 0
    lax.fori_loop(0, N // 16, _zero, 0)

    def _issue(k, b):
        for g in range(G):
            sv = srcall[pl.ds(k * KB + 16 * g, 16)]
            dv = dstall[pl.ds(k * KB + 16 * g, 16)]
            pltpu.async_copy(h_hbm.at[sv], rowS.at[b, pl.ds(16 * g, 16)],
                             semS.at[b])
            pltpu.async_copy(h_hbm.at[dv], rowD.at[b, pl.ds(16 * g, 16)],
                             semD.at[b])

    _issue(0, 0)

    def _batch(bk, _):
        b = lax.rem(bk, 2)

        @pl.when(bk + 1 < NBATCH)
        def _():
            _issue(bk + 1, 1 - b)

        pltpu.make_async_copy(h_hbm.at[pl.ds(0, KB)], rowS.at[b],
                              semS.at[b]).wait()
        pltpu.make_async_copy(h_hbm.at[pl.ds(0, KB)], rowD.at[b],
                              semD.at[b]).wait()

        io = lax.iota(jnp.int32, 16)

        @plsc.parallel_loop(0, KB, unroll=2)
        def _edge(e):
            a0 = jnp.zeros((16,), jnp.float32)
            a1 = jnp.zeros((16,), jnp.float32)
            for t in range(4):
                sa, sb = plsc.unpack(
                    plsc.bitcast(rowS[b, e, pl.ds(16 * t, 16)], jnp.bfloat16),
                    format=plsc.PackFormat.INTERLEAVED)
                da, db = plsc.unpack(
                    plsc.bitcast(rowD[b, e, pl.ds(16 * t, 16)], jnp.bfloat16),
                    format=plsc.PackFormat.INTERLEAVED)
                a0 += sa * da
                a1 += sb * db
            accbuf[pl.ds(e * 16, 16)] = a0 + a1

        for g in range(G):
            idxb = (io + 16 * g) * 16
            vals = [plsc.load_gather(accbuf, [idxb + l]) for l in range(16)]
            while len(vals) > 1:
                vals = [vals[i] + vals[i + 1] for i in range(0, len(vals), 2)]
            acc = vals[0]
            sv = srcall[pl.ds(bk * KB + 16 * g, 16)]
            dv = dstall[pl.ds(bk * KB + 16 * g, 16)]
            rns = plsc.load_gather(rnbt, [sv])
            rnd = plsc.load_gather(rnt, [dv])
            eev = jnp.exp(acc * rns * rnd)
            eeall[pl.ds(bk * KB + 16 * g, 16)] = eev

            kv, vv = plsc.sort_key_val(dv, eev)
            csum = plsc.cumsum(vv)
            knext = jnp.take_along_axis(kv, jnp.minimum(io + 1, 15), axis=0)
            last = (kv != knext) | (io == 15)
            kprev = jnp.take_along_axis(kv, jnp.maximum(io - 1, 0), axis=0)
            first = (kv != kprev) | (io == 0)
            sor = plsc.cummax(jnp.where(first, io, 0))
            prev_cs = jnp.take_along_axis(csum, jnp.maximum(sor - 1, 0),
                                          axis=0)
            runsum = csum - jnp.where(sor > 0, prev_cs, 0.0)
            plsc.addupdate_scatter(denloc, [kv], runsum, mask=last)
        return 0

    lax.fori_loop(0, NBATCH, _batch, 0)
    pltpu.sync_copy(eeall, ee_hbm.at[pl.ds(base, EPW)])
    pltpu.sync_copy(denloc, den_hbm.at[pl.ds(wid * N, N)])


KB2 = 32
G2 = KB2 // 16
NB2 = (EPW + KB2 - 1) // KB2
EPAD = NB2 * KB2
RB = 624


@functools.partial(
    pl.kernel,
    out_type=jax.ShapeDtypeStruct((NC, N, D), jnp.float32),
    mesh=_mesh,
    scratch_types=[
        pltpu.VMEM((EPAD,), jnp.int32),
        pltpu.VMEM((EPAD,), jnp.int32),
        pltpu.VMEM((EPAD,), jnp.float32),
        pltpu.VMEM((KB2,), jnp.float32),
        pltpu.VMEM((2, KB2, D), jnp.float32),
        pltpu.VMEM_SHARED((N, D), jnp.float32),
        pltpu.SemaphoreType.DMA((2,)),
        pltpu.SemaphoreType.DMA((2,)),
    ],
    compiler_params=pltpu.CompilerParams(needs_layout_passes=False),
)
def _sc_pass_b(h_hbm, src_hbm, dst_hbm, ee_hbm, zz_hbm,
               out_hbm,
               srcall, dstall, eeall, wbuf, rowS, spacc, semS, semW):
    c = lax.axis_index("c")
    s = lax.axis_index("s")
    wid = c * NS + s
    base = wid * EPW

    pltpu.sync_copy(src_hbm.at[pl.ds(base, EPW)], srcall.at[pl.ds(0, EPW)])
    pltpu.sync_copy(dst_hbm.at[pl.ds(base, EPW)], dstall.at[pl.ds(0, EPW)])
    pltpu.sync_copy(ee_hbm.at[pl.ds(base, EPW)], eeall.at[pl.ds(0, EPW)])
    srcall[pl.ds(EPW, EPAD - EPW)] = jnp.zeros((EPAD - EPW,), jnp.int32)
    dstall[pl.ds(EPW, EPAD - EPW)] = jnp.zeros((EPAD - EPW,), jnp.int32)

    nz = lax.select(s == NS - 1, 40, 39)
    rbase = s * RB

    def _zchunk(i, _):
        pltpu.sync_copy(zz_hbm, spacc.at[pl.ds(rbase + 16 * i, 16)])
        return 0
    lax.fori_loop(0, nz, _zchunk, 0)
    plsc.subcore_barrier()

    io = lax.iota(jnp.int32, 16)

    def _issue(k, b):
        for g in range(G2):
            sv = srcall[pl.ds(k * KB2 + 16 * g, 16)]
            pltpu.async_copy(h_hbm.at[sv], rowS.at[b, pl.ds(16 * g, 16)],
                             semS.at[b])

    _issue(0, 0)

    def _drain_scatter(b):
        for _ in range(G2):
            pltpu.make_async_copy(h_hbm.at[pl.ds(0, 16)],
                                  rowS.at[b, pl.ds(0, 16)],
                                  semW.at[b]).wait()

    def _batch(bk, _):
        b = lax.rem(bk, 2)

        @pl.when(bk >= 1)
        def _():
            _drain_scatter(1 - b)

        @pl.when(bk + 1 < NB2)
        def _():
            _issue(bk + 1, 1 - b)

        pltpu.make_async_copy(h_hbm.at[pl.ds(0, KB2)], rowS.at[b],
                              semS.at[b]).wait()

        for g in range(G2):
            off = bk * KB2 + 16 * g
            w = eeall[pl.ds(off, 16)]
            wbuf[pl.ds(16 * g, 16)] = jnp.where(off + io < EPW, w, 0.0)

        @plsc.parallel_loop(0, KB2, unroll=2)
        def _scale(e):
            wv = plsc.load_gather(wbuf, [jnp.full((16,), 0, jnp.int32) + e])
            for t in range(8):
                rowS[b, e, pl.ds(16 * t, 16)] = (
                    rowS[b, e, pl.ds(16 * t, 16)] * wv)

        for g in range(G2):
            dv = dstall[pl.ds(bk * KB2 + 16 * g, 16)]
            pltpu.async_copy(rowS.at[b, pl.ds(16 * g, 16)], spacc.at[dv],
                             semW.at[b], add=True)
        return 0

    lax.fori_loop(0, NB2, _batch, 0)
    _drain_scatter(lax.rem(NB2 - 1, 2))
    plsc.subcore_barrier()

    def _ochunk(i, _):
        pltpu.sync_copy(spacc.at[pl.ds(rbase + 16 * i, 16)],
                        out_hbm.at[c, pl.ds(rbase + 16 * i, 16)])
        return 0
    lax.fori_loop(0, nz, _ochunk, 0)


def kernel(x, edge_index, W1, b1, Wl1, bl1, Wl2, bl2, Wl3, bl3, Wl4, bl4,
           W4, b4, beta2, beta3, beta5, beta6):
    src = edge_index[0]
    dst = edge_index[1]
    zz = jnp.zeros((16, D), jnp.float32)

    h, lin, rn, rnb, se = _prep0(x, W1, b1, Wl1, bl1, beta2)

    layers = [(Wl2, bl2, beta3), (Wl3, bl3, beta5), (Wl4, bl4, beta6)]
    for i in range(4):
        hb = lax.bitcast_convert_type(
            h.astype(jnp.bfloat16).reshape(N, D // 2, 2), jnp.float32)
        ee, denp = _sc_pass_a(hb, src, dst, rnb.reshape(N), rn.reshape(N))
        outp = _sc_pass_b(h, src, dst, ee, zz)
        dnpT = denp.reshape(NW, N).T
        if i < 3:
            Wl, bl, beta = layers[i]
            h, lin, rn, rnb, se = _fused(outp, dnpT, se, h, lin, Wl, bl, beta)
        else:
            return _last(outp, dnpT, se, h, lin, W4, b4)


# speedup vs baseline: 1.2186x; 1.1665x over previous
import functools

import jax
import jax.numpy as jnp
from jax import lax
from jax.experimental import pallas as pl
from jax.experimental.pallas import tpu as pltpu
from jax.experimental.pallas import tpu_sc as plsc

N = 10000
E = 320000
D = 128
NC = 2
NS = 16
NW = NC * NS
EPW = E // NW
KB = 80
NBATCH = EPW // KB
G = KB // 16

_mesh = plsc.VectorSubcoreMesh(core_axis_name="c", subcore_axis_name="s")

ROW_BLK = 1000
GRID = N // ROW_BLK


def _dense_tail(h, Wl, bl, beta, lin_ref, rn_ref, rnb_ref, se_ref):
    lin_ref[...] = jnp.dot(h, Wl, preferred_element_type=jnp.float32) + bl
    ss = jnp.sum(h * h, axis=1, keepdims=True)
    nrm = jnp.maximum(jnp.sqrt(ss), 1e-12)
    rn = 1.0 / nrm
    rn_ref[...] = rn
    rnb_ref[...] = rn * beta
    se_ref[...] = jnp.exp(beta * (ss * rn * rn))


def _prep0_body(x_ref, W1_ref, b1_ref, Wl_ref, bl_ref, beta_ref,
                h_ref, lin_ref, rn_ref, rnb_ref, se_ref):
    h = jnp.dot(x_ref[...], W1_ref[...],
                preferred_element_type=jnp.float32) + b1_ref[...]
    h_ref[...] = h
    _dense_tail(h, Wl_ref[...], bl_ref[...], beta_ref[...],
                lin_ref, rn_ref, rnb_ref, se_ref)


def _prep0(x, W1, b1, Wl, bl, beta):
    return pl.pallas_call(
        _prep0_body,
        grid=(GRID,),
        in_specs=[
            pl.BlockSpec((ROW_BLK, D), lambda i: (i, 0)),
            pl.BlockSpec((D, D), lambda i: (0, 0)),
            pl.BlockSpec((1, D), lambda i: (0, 0)),
            pl.BlockSpec((D, D), lambda i: (0, 0)),
            pl.BlockSpec((1, D), lambda i: (0, 0)),
            pl.BlockSpec((1, 1), lambda i: (0, 0)),
        ],
        out_specs=[
            pl.BlockSpec((ROW_BLK, D), lambda i: (i, 0)),
            pl.BlockSpec((ROW_BLK, D), lambda i: (i, 0)),
            pl.BlockSpec((ROW_BLK, 1), lambda i: (i, 0)),
            pl.BlockSpec((ROW_BLK, 1), lambda i: (i, 0)),
            pl.BlockSpec((ROW_BLK, 1), lambda i: (i, 0)),
        ],
        out_shape=[
            jax.ShapeDtypeStruct((N, D), jnp.float32),
            jax.ShapeDtypeStruct((N, D), jnp.float32),
            jax.ShapeDtypeStruct((N, 1), jnp.float32),
            jax.ShapeDtypeStruct((N, 1), jnp.float32),
            jax.ShapeDtypeStruct((N, 1), jnp.float32),
        ],
    )(x, W1, b1.reshape(1, D), Wl, bl.reshape(1, D), beta.reshape(1, 1))


def _mid_body(den_ref, se_ref, rd_ref, sw_ref):
    dsum = jnp.sum(den_ref[...], axis=0, keepdims=True) + se_ref[...]
    rd = 1.0 / (dsum + 1e-16)
    rd_ref[...] = rd
    sw_ref[...] = se_ref[...] * rd


def _mid(denp, se):
    return pl.pallas_call(
        _mid_body,
        out_shape=[
            jax.ShapeDtypeStruct((1, N), jnp.float32),
            jax.ShapeDtypeStruct((1, N), jnp.float32),
        ],
    )(denp, se.reshape(1, N))


def _fused_body(o0_ref, o1_ref, dnp_ref, se_ref0, h_ref, lin_ref, Wl_ref,
                bl_ref, beta_ref, h2_ref, lin2_ref, rn_ref, rnb_ref, se_ref):
    se0 = se_ref0[...]
    rd = 1.0 / (jnp.sum(dnp_ref[...], axis=1, keepdims=True) + se0)
    z = ((o0_ref[...] + o1_ref[...]) * rd + (se0 * rd) * h_ref[...]
         + lin_ref[...])
    h = jnp.where(z > 0, z, jnp.exp(z) - 1.0)
    h2_ref[...] = h
    _dense_tail(h, Wl_ref[...], bl_ref[...], beta_ref[...],
                lin2_ref, rn_ref, rnb_ref, se_ref)


def _fused(outp, dnpT, se, h, lin, Wl, bl, beta):
    return pl.pallas_call(
        _fused_body,
        grid=(GRID,),
        in_specs=[
            pl.BlockSpec((ROW_BLK, D), lambda i: (i, 0)),
            pl.BlockSpec((ROW_BLK, D), lambda i: (i, 0)),
            pl.BlockSpec((ROW_BLK, NW), lambda i: (i, 0)),
            pl.BlockSpec((ROW_BLK, 1), lambda i: (i, 0)),
            pl.BlockSpec((ROW_BLK, D), lambda i: (i, 0)),
            pl.BlockSpec((ROW_BLK, D), lambda i: (i, 0)),
            pl.BlockSpec((D, D), lambda i: (0, 0)),
            pl.BlockSpec((1, D), lambda i: (0, 0)),
            pl.BlockSpec((1, 1), lambda i: (0, 0)),
        ],
        out_specs=[
            pl.BlockSpec((ROW_BLK, D), lambda i: (i, 0)),
            pl.BlockSpec((ROW_BLK, D), lambda i: (i, 0)),
            pl.BlockSpec((ROW_BLK, 1), lambda i: (i, 0)),
            pl.BlockSpec((ROW_BLK, 1), lambda i: (i, 0)),
            pl.BlockSpec((ROW_BLK, 1), lambda i: (i, 0)),
        ],
        out_shape=[
            jax.ShapeDtypeStruct((N, D), jnp.float32),
            jax.ShapeDtypeStruct((N, D), jnp.float32),
            jax.ShapeDtypeStruct((N, 1), jnp.float32),
            jax.ShapeDtypeStruct((N, 1), jnp.float32),
            jax.ShapeDtypeStruct((N, 1), jnp.float32),
        ],
    )(outp[0], outp[1], dnpT,
      se, h, lin, Wl, bl.reshape(1, D), beta.reshape(1, 1))


def _last_body(o0_ref, o1_ref, dnp_ref, se_ref0, h_ref, lin_ref, W4_ref,
               b4_ref, out_ref):
    se0 = se_ref0[...]
    rd = 1.0 / (jnp.sum(dnp_ref[...], axis=1, keepdims=True) + se0)
    z = ((o0_ref[...] + o1_ref[...]) * rd + (se0 * rd) * h_ref[...]
         + lin_ref[...])
    h = jnp.where(z > 0, z, jnp.exp(z) - 1.0)
    out_ref[...] = jnp.dot(h, W4_ref[...],
                           preferred_element_type=jnp.float32) + b4_ref[...]


def _last(outp, dnpT, se, h, lin, W4, b4):
    C = W4.shape[1]
    return pl.pallas_call(
        _last_body,
        grid=(GRID,),
        in_specs=[
            pl.BlockSpec((ROW_BLK, D), lambda i: (i, 0)),
            pl.BlockSpec((ROW_BLK, D), lambda i: (i, 0)),
            pl.BlockSpec((ROW_BLK, NW), lambda i: (i, 0)),
            pl.BlockSpec((ROW_BLK, 1), lambda i: (i, 0)),
            pl.BlockSpec((ROW_BLK, D), lambda i: (i, 0)),
            pl.BlockSpec((ROW_BLK, D), lambda i: (i, 0)),
            pl.BlockSpec((D, C), lambda i: (0, 0)),
            pl.BlockSpec((1, C), lambda i: (0, 0)),
        ],
        out_specs=pl.BlockSpec((ROW_BLK, C), lambda i: (i, 0)),
        out_shape=jax.ShapeDtypeStruct((N, C), jnp.float32),
    )(outp[0], outp[1], dnpT,
      se, h, lin, W4, b4.reshape(1, C))


@functools.partial(
    pl.kernel,
    out_type=(
        jax.ShapeDtypeStruct((E,), jnp.float32),
        jax.ShapeDtypeStruct((NW * N,), jnp.float32),
    ),
    mesh=_mesh,
    scratch_types=[
        pltpu.VMEM((EPW,), jnp.int32),
        pltpu.VMEM((EPW,), jnp.int32),
        pltpu.VMEM((N,), jnp.float32),
        pltpu.VMEM((N,), jnp.float32),
        pltpu.VMEM((N,), jnp.float32),
        pltpu.VMEM((EPW,), jnp.float32),
        pltpu.VMEM((2, KB, D // 2), jnp.float32),
        pltpu.VMEM((2, KB, D // 2), jnp.float32),
        pltpu.VMEM((KB * 16,), jnp.float32),
        pltpu.SemaphoreType.DMA((2,)),
        pltpu.SemaphoreType.DMA((2,)),
    ],
    compiler_params=pltpu.CompilerParams(needs_layout_passes=False,
                                         use_tc_tiling_on_sc=False),
)
def _sc_pass_a(h_hbm, src_hbm, dst_hbm, rnb_hbm, rn_hbm,
               ee_hbm, den_hbm,
               srcall, dstall, rnbt, rnt, denloc, eeall, rowS, rowD, accbuf,
               semS, semD):
    c = lax.axis_index("c")
    s = lax.axis_index("s")
    wid = c * NS + s
    base = wid * EPW

    pltpu.sync_copy(src_hbm.at[pl.ds(base, EPW)], srcall)
    pltpu.sync_copy(dst_hbm.at[pl.ds(base, EPW)], dstall)
    pltpu.sync_copy(rnb_hbm, rnbt)
    pltpu.sync_copy(rn_hbm, rnt)

    def _zero(i, _):
        denloc[pl.ds(i * 16, 16)] = jnp.zeros((16,), jnp.float32)
        return 0
    lax.fori_loop(0, N // 16, _zero, 0)

    def _issue(k, b):
        for g in range(G):
            sv = srcall[pl.ds(k * KB + 16 * g, 16)]
            dv = dstall[pl.ds(k * KB + 16 * g, 16)]
            pltpu.async_copy(h_hbm.at[sv], rowS.at[b, pl.ds(16 * g, 16)],
                             semS.at[b])
            pltpu.async_copy(h_hbm.at[dv], rowD.at[b, pl.ds(16 * g, 16)],
                             semD.at[b])

    _issue(0, 0)

    def _batch(bk, _):
        b = lax.rem(bk, 2)

        @pl.when(bk + 1 < NBATCH)
        def _():
            _issue(bk + 1, 1 - b)

        pltpu.make_async_copy(h_hbm.at[pl.ds(0, KB)], rowS.at[b],
                              semS.at[b]).wait()
        pltpu.make_async_copy(h_hbm.at[pl.ds(0, KB)], rowD.at[b],
                              semD.at[b]).wait()

        io = lax.iota(jnp.int32, 16)

        @plsc.parallel_loop(0, KB, unroll=2)
        def _edge(e):
            a0 = jnp.zeros((16,), jnp.float32)
            a1 = jnp.zeros((16,), jnp.float32)
            for t in range(4):
                sa, sb = plsc.unpack(
                    plsc.bitcast(rowS[b, e, pl.ds(16 * t, 16)], jnp.bfloat16),
                    format=plsc.PackFormat.INTERLEAVED)
                da, db = plsc.unpack(
                    plsc.bitcast(rowD[b, e, pl.ds(16 * t, 16)], jnp.bfloat16),
                    format=plsc.PackFormat.INTERLEAVED)
                a0 += sa * da
                a1 += sb * db
            accbuf[pl.ds(e * 16, 16)] = a0 + a1

        for g in range(G):
            idxb = (io + 16 * g) * 16
            vals = [plsc.load_gather(accbuf, [idxb + l]) for l in range(16)]
            while len(vals) > 1:
                vals = [vals[i] + vals[i + 1] for i in range(0, len(vals), 2)]
            acc = vals[0]
            sv = srcall[pl.ds(bk * KB + 16 * g, 16)]
            dv = dstall[pl.ds(bk * KB + 16 * g, 16)]
            rns = plsc.load_gather(rnbt, [sv])
            rnd = plsc.load_gather(rnt, [dv])
            eev = jnp.exp(acc * rns * rnd)
            eeall[pl.ds(bk * KB + 16 * g, 16)] = eev

            kv, vv = plsc.sort_key_val(dv, eev)
            csum = plsc.cumsum(vv)
            knext = jnp.take_along_axis(kv, jnp.minimum(io + 1, 15), axis=0)
            last = (kv != knext) | (io == 15)
            kprev = jnp.take_along_axis(kv, jnp.maximum(io - 1, 0), axis=0)
            first = (kv != kprev) | (io == 0)
            sor = plsc.cummax(jnp.where(first, io, 0))
            prev_cs = jnp.take_along_axis(csum, jnp.maximum(sor - 1, 0),
                                          axis=0)
            runsum = csum - jnp.where(sor > 0, prev_cs, 0.0)
            plsc.addupdate_scatter(denloc, [kv], runsum, mask=last)
        return 0

    lax.fori_loop(0, NBATCH, _batch, 0)
    pltpu.sync_copy(eeall, ee_hbm.at[pl.ds(base, EPW)])
    pltpu.sync_copy(denloc, den_hbm.at[pl.ds(wid * N, N)])


KB2 = 80
G2 = KB2 // 16
NB2 = (EPW + KB2 - 1) // KB2
EPAD = NB2 * KB2
RB = 624


@functools.partial(
    pl.kernel,
    out_type=jax.ShapeDtypeStruct((NC, N, D), jnp.float32),
    mesh=_mesh,
    scratch_types=[
        pltpu.VMEM((EPAD,), jnp.int32),
        pltpu.VMEM((EPAD,), jnp.int32),
        pltpu.VMEM((EPAD,), jnp.float32),
        pltpu.VMEM((KB2,), jnp.float32),
        pltpu.VMEM((2, KB2, D), jnp.float32),
        pltpu.VMEM_SHARED((N, D), jnp.float32),
        pltpu.SemaphoreType.DMA((2,)),
        pltpu.SemaphoreType.DMA((2,)),
    ],
    compiler_params=pltpu.CompilerParams(needs_layout_passes=False),
)
def _sc_pass_b(h_hbm, src_hbm, dst_hbm, ee_hbm, zz_hbm,
               out_hbm,
               srcall, dstall, eeall, wbuf, rowS, spacc, semS, semW):
    c = lax.axis_index("c")
    s = lax.axis_index("s")
    wid = c * NS + s
    base = wid * EPW

    pltpu.sync_copy(src_hbm.at[pl.ds(base, EPW)], srcall.at[pl.ds(0, EPW)])
    pltpu.sync_copy(dst_hbm.at[pl.ds(base, EPW)], dstall.at[pl.ds(0, EPW)])
    pltpu.sync_copy(ee_hbm.at[pl.ds(base, EPW)], eeall.at[pl.ds(0, EPW)])
    if EPAD > EPW:
        srcall[pl.ds(EPW, EPAD - EPW)] = jnp.zeros((EPAD - EPW,), jnp.int32)
        dstall[pl.ds(EPW, EPAD - EPW)] = jnp.zeros((EPAD - EPW,), jnp.int32)

    nz = lax.select(s == NS - 1, 40, 39)
    rbase = s * RB

    def _zchunk(i, _):
        pltpu.sync_copy(zz_hbm, spacc.at[pl.ds(rbase + 16 * i, 16)])
        return 0
    lax.fori_loop(0, nz, _zchunk, 0)
    plsc.subcore_barrier()

    io = lax.iota(jnp.int32, 16)

    def _issue(k, b):
        for g in range(G2):
            sv = srcall[pl.ds(k * KB2 + 16 * g, 16)]
            pltpu.async_copy(h_hbm.at[sv], rowS.at[b, pl.ds(16 * g, 16)],
                             semS.at[b])

    _issue(0, 0)

    def _drain_scatter(b):
        for _ in range(G2):
            pltpu.make_async_copy(h_hbm.at[pl.ds(0, 16)],
                                  rowS.at[b, pl.ds(0, 16)],
                                  semW.at[b]).wait()

    def _batch(bk, _):
        b = lax.rem(bk, 2)

        @pl.when(bk >= 1)
        def _():
            _drain_scatter(1 - b)

        @pl.when(bk + 1 < NB2)
        def _():
            _issue(bk + 1, 1 - b)

        pltpu.make_async_copy(h_hbm.at[pl.ds(0, KB2)], rowS.at[b],
                              semS.at[b]).wait()

        for g in range(G2):
            off = bk * KB2 + 16 * g
            w = eeall[pl.ds(off, 16)]
            wbuf[pl.ds(16 * g, 16)] = jnp.where(off + io < EPW, w, 0.0)

        @plsc.parallel_loop(0, KB2, unroll=2)
        def _scale(e):
            wv = plsc.load_gather(wbuf, [jnp.full((16,), 0, jnp.int32) + e])
            for t in range(8):
                rowS[b, e, pl.ds(16 * t, 16)] = (
                    rowS[b, e, pl.ds(16 * t, 16)] * wv)

        for g in range(G2):
            dv = dstall[pl.ds(bk * KB2 + 16 * g, 16)]
            pltpu.async_copy(rowS.at[b, pl.ds(16 * g, 16)], spacc.at[dv],
                             semW.at[b], add=True)
        return 0

    lax.fori_loop(0, NB2, _batch, 0)
    _drain_scatter(lax.rem(NB2 - 1, 2))
    plsc.subcore_barrier()

    def _ochunk(i, _):
        pltpu.sync_copy(spacc.at[pl.ds(rbase + 16 * i, 16)],
                        out_hbm.at[c, pl.ds(rbase + 16 * i, 16)])
        return 0
    lax.fori_loop(0, nz, _ochunk, 0)


def kernel(x, edge_index, W1, b1, Wl1, bl1, Wl2, bl2, Wl3, bl3, Wl4, bl4,
           W4, b4, beta2, beta3, beta5, beta6):
    src = edge_index[0]
    dst = edge_index[1]
    zz = jnp.zeros((16, D), jnp.float32)

    h, lin, rn, rnb, se = _prep0(x, W1, b1, Wl1, bl1, beta2)

    layers = [(Wl2, bl2, beta3), (Wl3, bl3, beta5), (Wl4, bl4, beta6)]
    for i in range(4):
        hb = lax.bitcast_convert_type(
            h.astype(jnp.bfloat16).reshape(N, D // 2, 2), jnp.float32)
        ee, denp = _sc_pass_a(hb, src, dst, rnb.reshape(N), rn.reshape(N))
        outp = _sc_pass_b(h, src, dst, ee, zz)
        dnpT = denp.reshape(NW, N).T
        if i < 3:
            Wl, bl, beta = layers[i]
            h, lin, rn, rnb, se = _fused(outp, dnpT, se, h, lin, Wl, bl, beta)
        else:
            return _last(outp, dnpT, se, h, lin, W4, b4)
